# Initial kernel scaffold; baseline (speedup 1.0000x reference)
import jax
import jax.numpy as jnp
from jax.experimental import pallas as pl


def kernel(x, country_table, occupation_table, bn_gamma, bn_beta, W1, b1, W2, b2, W3, b3):
    raise NotImplementedError("write your pallas kernel here")


# speedup vs baseline: 2.6584x; 2.6584x over previous
import functools

import jax
import jax.numpy as jnp
from jax import lax
from jax.experimental import pallas as pl
from jax.experimental.pallas import tpu as pltpu
from jax.experimental.pallas import tpu_sc as plsc

B = 16384
NC, NS = 2, 16
NW = NC * NS
BPW = B // NW
GSZ = 16
NGRP = BPW // GSZ
TROWS = 80
EPS = 1e-5


def _sc_gather_body(tabc_hbm, tabo_hbm, ic_hbm, io_hbm, outc_hbm, outo_hbm,
                    tcv, tov, icv, iov, ocv, oov, sem):
  wid = lax.axis_index("s") * NC + lax.axis_index("c")
  base = wid * BPW
  cps = [
      pltpu.async_copy(tabc_hbm, tcv, sem),
      pltpu.async_copy(tabo_hbm, tov, sem),
      pltpu.async_copy(ic_hbm.at[pl.ds(base, BPW)], icv, sem),
      pltpu.async_copy(io_hbm.at[pl.ds(base, BPW)], iov, sem),
  ]
  for cp in cps:
    cp.wait()

  def group(g, carry):
    pc = icv[pl.ds(g * GSZ, GSZ)]
    po = iov[pl.ds(g * GSZ, GSZ)]
    for d in range(10):
      ocv[d, pl.ds(g * GSZ, GSZ)] = plsc.load_gather(tcv, [pc + d])
      oov[d, pl.ds(g * GSZ, GSZ)] = plsc.load_gather(tov, [po + d])
    return carry

  lax.fori_loop(0, NGRP, group, 0)

  zeros = jnp.zeros((GSZ,), jnp.float32)
  for d in range(10, 16):
    for k in range(NGRP):
      ocv[d, pl.ds(k * GSZ, GSZ)] = zeros
      oov[d, pl.ds(k * GSZ, GSZ)] = zeros

  pltpu.sync_copy(ocv, outc_hbm.at[:, pl.ds(base, BPW)])
  pltpu.sync_copy(oov, outo_hbm.at[:, pl.ds(base, BPW)])


def _sc_gather(tab_c, tab_o, idx_c, idx_o):
  mesh = plsc.VectorSubcoreMesh(core_axis_name="c", subcore_axis_name="s")
  fn = functools.partial(
      pl.kernel,
      out_type=(jax.ShapeDtypeStruct((16, B), jnp.float32),
                jax.ShapeDtypeStruct((16, B), jnp.float32)),
      mesh=mesh,
      scratch_types=[
          pltpu.VMEM((TROWS * 128,), jnp.float32),
          pltpu.VMEM((TROWS * 128,), jnp.float32),
          pltpu.VMEM((BPW,), jnp.int32),
          pltpu.VMEM((BPW,), jnp.int32),
          pltpu.VMEM((16, BPW), jnp.float32),
          pltpu.VMEM((16, BPW), jnp.float32),
          pltpu.SemaphoreType.DMA,
      ],
      compiler_params=pltpu.CompilerParams(needs_layout_passes=False),
  )(_sc_gather_body)
  return fn(tab_c, tab_o, idx_c, idx_o)


def _tc_dense_body(ct_ref, ot_ref, tt_ref, gc, bc, go, bo, gt, bt,
                   w1c, w1o, w1t, b1, w2, b2, w3, b3, out_ref):
  inv_b = 1.0 / B

  def bn(xp, g, b):
    m = jnp.sum(xp, axis=1, keepdims=True) * inv_b
    d = xp - m
    v = jnp.sum(d * d, axis=1, keepdims=True) * inv_b
    s = g[...] * lax.rsqrt(v + EPS)
    return d * s + b[...]

  xc = bn(ct_ref[...], gc, bc)
  xo = bn(ot_ref[...], go, bo)
  xt = bn(tt_ref[...], gt, bt)
  y1 = (jnp.dot(w1c[...], xc, preferred_element_type=jnp.float32)
        + jnp.dot(w1o[...], xo, preferred_element_type=jnp.float32)
        + jnp.dot(w1t[...], xt, preferred_element_type=jnp.float32)
        + b1[...])
  h1 = jnp.maximum(y1, 0.0)
  h2 = jnp.maximum(
      jnp.dot(w2[...], h1, preferred_element_type=jnp.float32) + b2[...], 0.0)
  y3 = jnp.dot(w3[...], h2, preferred_element_type=jnp.float32) + b3[...]
  out_ref[...] = jnp.transpose(y3)[:, 0:4]


def _tc_dense(ct, ot, tt, gc, bc, go, bo, gt, bt,
              w1c, w1o, w1t, b1, w2, b2, w3, b3, interpret=False):
  return pl.pallas_call(
      _tc_dense_body,
      out_shape=jax.ShapeDtypeStruct((B, 4), jnp.float32),
      interpret=interpret,
  )(ct, ot, tt, gc, bc, go, bo, gt, bt, w1c, w1o, w1t, b1, w2, b2, w3, b3)


def _pad2(a, rows, cols):
  r, c = a.shape
  return jnp.pad(a, ((0, rows - r), (0, cols - c)))


def _flat_table(tab):
  return jnp.pad(tab, ((0, 24), (0, 0))).reshape(TROWS * 128)


def kernel(x, country_table, occupation_table, bn_gamma, bn_beta,
           W1, b1, W2, b2, W3, b3):
  idx_c = (x[:, 1].astype(jnp.int32)) * 10
  idx_o = (x[:, 8].astype(jnp.int32)) * 10
  other_t = jnp.pad(
      jnp.stack([x[:, 0], x[:, 3], x[:, 4], x[:, 5], x[:, 6], x[:, 7]],
                axis=0), ((0, 2), (0, 0)))
  tab_c = _flat_table(country_table)
  tab_o = _flat_table(occupation_table)

  gc = _pad2(bn_gamma[0:10].reshape(10, 1), 16, 1)
  go = _pad2(bn_gamma[10:20].reshape(10, 1), 16, 1)
  gt = _pad2(bn_gamma[20:26].reshape(6, 1), 8, 1)
  bc = _pad2(bn_beta[0:10].reshape(10, 1), 16, 1)
  bo = _pad2(bn_beta[10:20].reshape(10, 1), 16, 1)
  bt = _pad2(bn_beta[20:26].reshape(6, 1), 8, 1)

  w1c = _pad2(W1[:, 0:10], 32, 16)
  w1o = _pad2(W1[:, 10:20], 32, 16)
  w1t = _pad2(W1[:, 20:26], 32, 8)
  w3p = _pad2(W3, 8, 16)
  b1r = b1.reshape(32, 1)
  b2r = b2.reshape(16, 1)
  b3r = _pad2(b3.reshape(4, 1), 8, 1)

  ct, ot = _sc_gather(tab_c, tab_o, idx_c, idx_o)
  return _tc_dense(ct, ot, other_t, gc, bc, go, bo, gt, bt,
                   w1c, w1o, w1t, b1r, W2, b2r, w3p, b3r)


# speedup vs baseline: 2.9042x; 1.0924x over previous
import functools

import jax
import jax.numpy as jnp
from jax import lax
from jax.experimental import pallas as pl
from jax.experimental.pallas import tpu as pltpu
from jax.experimental.pallas import tpu_sc as plsc

B = 16384
NC, NS = 2, 16
NW = NC * NS
BPW = B // NW
GSZ = 16
NGRP = BPW // GSZ
TFLAT = 10240
EPS = 1e-5
OTHER_COLS = (0, 3, 4, 5, 6, 7)


def _sc_gather_body(x_hbm, tabc_hbm, tabo_hbm, outc_hbm, outo_hbm, outt_hbm,
                    xv, tcv, tov, ocv, oov, otv, sem):
  wid = lax.axis_index("s") * NC + lax.axis_index("c")
  base = wid * BPW
  cps = [
      pltpu.async_copy(tabc_hbm, tcv, sem),
      pltpu.async_copy(tabo_hbm, tov, sem),
      pltpu.async_copy(x_hbm.at[pl.ds(base, BPW)], xv, sem),
  ]
  for cp in cps:
    cp.wait()

  iota = lax.iota(jnp.int32, GSZ)

  @plsc.parallel_loop(0, NGRP, unroll=2)
  def _group(g):
    rows = g * GSZ + iota
    sl = pl.ds(g * GSZ, GSZ)
    ic = plsc.load_gather(xv, [rows, jnp.full((GSZ,), 1, jnp.int32)])
    io = plsc.load_gather(xv, [rows, jnp.full((GSZ,), 8, jnp.int32)])
    pc = ic.astype(jnp.int32) * 10
    po = io.astype(jnp.int32) * 10
    for d in range(10):
      ocv[d, sl] = plsc.load_gather(tcv, [pc + d])
      oov[d, sl] = plsc.load_gather(tov, [po + d])
    for k, c in enumerate(OTHER_COLS):
      otv[k, sl] = plsc.load_gather(
          xv, [rows, jnp.full((GSZ,), c, jnp.int32)])

  zeros = jnp.zeros((GSZ,), jnp.float32)
  for k in range(NGRP):
    sl = pl.ds(k * GSZ, GSZ)
    for d in range(10, 16):
      ocv[d, sl] = zeros
      oov[d, sl] = zeros
    for d in range(6, 8):
      otv[d, sl] = zeros

  pltpu.sync_copy(ocv, outc_hbm.at[:, pl.ds(base, BPW)])
  pltpu.sync_copy(oov, outo_hbm.at[:, pl.ds(base, BPW)])
  pltpu.sync_copy(otv, outt_hbm.at[:, pl.ds(base, BPW)])


def _sc_gather(x, tab_c, tab_o):
  mesh = plsc.VectorSubcoreMesh(core_axis_name="c", subcore_axis_name="s")
  fn = functools.partial(
      pl.kernel,
      out_type=(jax.ShapeDtypeStruct((16, B), jnp.float32),
                jax.ShapeDtypeStruct((16, B), jnp.float32),
                jax.ShapeDtypeStruct((8, B), jnp.float32)),
      mesh=mesh,
      scratch_types=[
          pltpu.VMEM((BPW, 10), jnp.float32),
          pltpu.VMEM((TFLAT,), jnp.float32),
          pltpu.VMEM((TFLAT,), jnp.float32),
          pltpu.VMEM((16, BPW), jnp.float32),
          pltpu.VMEM((16, BPW), jnp.float32),
          pltpu.VMEM((8, BPW), jnp.float32),
          pltpu.SemaphoreType.DMA,
      ],
      compiler_params=pltpu.CompilerParams(needs_layout_passes=False),
  )(_sc_gather_body)
  return fn(x, tab_c, tab_o)


def _tc_dense_body(ct_ref, ot_ref, tt_ref, gc, bc, go, bo, gt, bt,
                   w1c, w1o, w1t, b1, w2, b2, w3, b3, out_ref):
  inv_b = 1.0 / B

  def bn(xp, g, b):
    m = jnp.sum(xp, axis=1, keepdims=True) * inv_b
    d = xp - m
    v = jnp.sum(d * d, axis=1, keepdims=True) * inv_b
    s = g[...] * lax.rsqrt(v + EPS)
    return d * s + b[...]

  xc = bn(ct_ref[...], gc, bc)
  xo = bn(ot_ref[...], go, bo)
  xt = bn(tt_ref[...], gt, bt)
  y1 = (jnp.dot(w1c[...], xc, preferred_element_type=jnp.float32)
        + jnp.dot(w1o[...], xo, preferred_element_type=jnp.float32)
        + jnp.dot(w1t[...], xt, preferred_element_type=jnp.float32)
        + b1[...])
  h1 = jnp.maximum(y1, 0.0)
  h2 = jnp.maximum(
      jnp.dot(w2[...], h1, preferred_element_type=jnp.float32) + b2[...], 0.0)
  y3 = jnp.dot(w3[...], h2, preferred_element_type=jnp.float32) + b3[...]
  out_ref[...] = jnp.transpose(y3)[:, 0:4]


def _tc_dense(ct, ot, tt, gc, bc, go, bo, gt, bt,
              w1c, w1o, w1t, b1, w2, b2, w3, b3, interpret=False):
  return pl.pallas_call(
      _tc_dense_body,
      out_shape=jax.ShapeDtypeStruct((B, 4), jnp.float32),
      interpret=interpret,
  )(ct, ot, tt, gc, bc, go, bo, gt, bt, w1c, w1o, w1t, b1, w2, b2, w3, b3)


def _pad2(a, rows, cols):
  r, c = a.shape
  return jnp.pad(a, ((0, rows - r), (0, cols - c)))


def _flat_table(tab):
  return jnp.pad(tab, ((0, 24), (0, 0))).reshape(TFLAT)


def kernel(x, country_table, occupation_table, bn_gamma, bn_beta,
           W1, b1, W2, b2, W3, b3):
  tab_c = _flat_table(country_table)
  tab_o = _flat_table(occupation_table)

  gc = _pad2(bn_gamma[0:10].reshape(10, 1), 16, 1)
  go = _pad2(bn_gamma[10:20].reshape(10, 1), 16, 1)
  gt = _pad2(bn_gamma[20:26].reshape(6, 1), 8, 1)
  bc = _pad2(bn_beta[0:10].reshape(10, 1), 16, 1)
  bo = _pad2(bn_beta[10:20].reshape(10, 1), 16, 1)
  bt = _pad2(bn_beta[20:26].reshape(6, 1), 8, 1)

  w1c = _pad2(W1[:, 0:10], 32, 16)
  w1o = _pad2(W1[:, 10:20], 32, 16)
  w1t = _pad2(W1[:, 20:26], 32, 8)
  w3p = _pad2(W3, 8, 16)
  b1r = b1.reshape(32, 1)
  b2r = b2.reshape(16, 1)
  b3r = _pad2(b3.reshape(4, 1), 8, 1)

  ct, ot, tt = _sc_gather(x, tab_c, tab_o)
  return _tc_dense(ct, ot, tt, gc, bc, go, bo, gt, bt,
                   w1c, w1o, w1t, b1r, W2, b2r, w3p, b3r)


# speedup vs baseline: 3.8153x; 1.3137x over previous
import functools

import jax
import jax.numpy as jnp
from jax import lax
from jax.experimental import pallas as pl
from jax.experimental.pallas import tpu as pltpu
from jax.experimental.pallas import tpu_sc as plsc

B = 16384
NC, NS = 2, 16
NW = NC * NS
BPW = B // NW
GSZ = 16
NGRP = BPW // GSZ
TFLAT = 10240
EPS = 1e-5


def _sc_gather_body(tabc_hbm, tabo_hbm, ic_hbm, io_hbm, outc_hbm, outo_hbm,
                    tcv, tov, icv, iov, ocv, oov, sem):
  wid = lax.axis_index("s") * NC + lax.axis_index("c")
  base = wid * BPW
  cps = [
      pltpu.async_copy(tabc_hbm, tcv, sem),
      pltpu.async_copy(tabo_hbm, tov, sem),
      pltpu.async_copy(ic_hbm.at[pl.ds(base, BPW)], icv, sem),
      pltpu.async_copy(io_hbm.at[pl.ds(base, BPW)], iov, sem),
  ]
  for cp in cps:
    cp.wait()

  @plsc.parallel_loop(0, NGRP, unroll=2)
  def _group(g):
    sl = pl.ds(g * GSZ, GSZ)
    pc = icv[sl]
    po = iov[sl]
    for d in range(10):
      ocv[d, sl] = plsc.load_gather(tcv, [pc + d])
      oov[d, sl] = plsc.load_gather(tov, [po + d])

  zeros = jnp.zeros((GSZ,), jnp.float32)
  for k in range(NGRP):
    sl = pl.ds(k * GSZ, GSZ)
    for d in range(10, 16):
      ocv[d, sl] = zeros
      oov[d, sl] = zeros

  pltpu.sync_copy(ocv, outc_hbm.at[:, pl.ds(base, BPW)])
  pltpu.sync_copy(oov, outo_hbm.at[:, pl.ds(base, BPW)])


def _sc_gather(tab_c, tab_o, idx_c, idx_o):
  mesh = plsc.VectorSubcoreMesh(core_axis_name="c", subcore_axis_name="s")
  fn = functools.partial(
      pl.kernel,
      out_type=(jax.ShapeDtypeStruct((16, B), jnp.float32),
                jax.ShapeDtypeStruct((16, B), jnp.float32)),
      mesh=mesh,
      scratch_types=[
          pltpu.VMEM((TFLAT,), jnp.float32),
          pltpu.VMEM((TFLAT,), jnp.float32),
          pltpu.VMEM((BPW,), jnp.int32),
          pltpu.VMEM((BPW,), jnp.int32),
          pltpu.VMEM((16, BPW), jnp.float32),
          pltpu.VMEM((16, BPW), jnp.float32),
          pltpu.SemaphoreType.DMA,
      ],
      compiler_params=pltpu.CompilerParams(needs_layout_passes=False),
  )(_sc_gather_body)
  return fn(tab_c, tab_o, idx_c, idx_o)


def _tc_dense_body(ct_ref, ot_ref, tt_ref, gc, bc, go, bo, gt, bt,
                   w1c, w1o, w1t, b1, w2, b2, w3, b3, out_ref):
  inv_b = 1.0 / B

  def bn(xp, g, b):
    m = jnp.sum(xp, axis=1, keepdims=True) * inv_b
    d = xp - m
    v = jnp.sum(d * d, axis=1, keepdims=True) * inv_b
    s = g[...] * lax.rsqrt(v + EPS)
    return d * s + b[...]

  xc = bn(ct_ref[...], gc, bc)
  xo = bn(ot_ref[...], go, bo)
  xt = bn(tt_ref[...], gt, bt)
  y1 = (jnp.dot(w1c[...], xc, preferred_element_type=jnp.float32)
        + jnp.dot(w1o[...], xo, preferred_element_type=jnp.float32)
        + jnp.dot(w1t[...], xt, preferred_element_type=jnp.float32)
        + b1[...])
  h1 = jnp.maximum(y1, 0.0)
  h2 = jnp.maximum(
      jnp.dot(w2[...], h1, preferred_element_type=jnp.float32) + b2[...], 0.0)
  out_ref[...] = jnp.dot(w3[...], h2, preferred_element_type=jnp.float32) + b3[...]


def _tc_dense(ct, ot, tt, gc, bc, go, bo, gt, bt,
              w1c, w1o, w1t, b1, w2, b2, w3, b3, interpret=False):
  return pl.pallas_call(
      _tc_dense_body,
      out_shape=jax.ShapeDtypeStruct((8, B), jnp.float32),
      interpret=interpret,
  )(ct, ot, tt, gc, bc, go, bo, gt, bt, w1c, w1o, w1t, b1, w2, b2, w3, b3)


def _pad2(a, rows, cols):
  r, c = a.shape
  return jnp.pad(a, ((0, rows - r), (0, cols - c)))


def _flat_table(tab):
  return jnp.pad(tab, ((0, 24), (0, 0))).reshape(TFLAT)


def kernel(x, country_table, occupation_table, bn_gamma, bn_beta,
           W1, b1, W2, b2, W3, b3):
  xt_all = x.T
  idx_c = xt_all[1].astype(jnp.int32) * 10
  idx_o = xt_all[8].astype(jnp.int32) * 10
  other_t = jnp.pad(xt_all[jnp.array([0, 3, 4, 5, 6, 7])], ((0, 2), (0, 0)))

  tab_c = _flat_table(country_table)
  tab_o = _flat_table(occupation_table)

  gc = _pad2(bn_gamma[0:10].reshape(10, 1), 16, 1)
  go = _pad2(bn_gamma[10:20].reshape(10, 1), 16, 1)
  gt = _pad2(bn_gamma[20:26].reshape(6, 1), 8, 1)
  bc = _pad2(bn_beta[0:10].reshape(10, 1), 16, 1)
  bo = _pad2(bn_beta[10:20].reshape(10, 1), 16, 1)
  bt = _pad2(bn_beta[20:26].reshape(6, 1), 8, 1)

  w1c = _pad2(W1[:, 0:10], 32, 16)
  w1o = _pad2(W1[:, 10:20], 32, 16)
  w1t = _pad2(W1[:, 20:26], 32, 8)
  w3p = _pad2(W3, 8, 16)
  b1r = b1.reshape(32, 1)
  b2r = b2.reshape(16, 1)
  b3r = _pad2(b3.reshape(4, 1), 8, 1)

  ct, ot = _sc_gather(tab_c, tab_o, idx_c, idx_o)
  y3 = _tc_dense(ct, ot, other_t, gc, bc, go, bo, gt, bt,
                 w1c, w1o, w1t, b1r, W2, b2r, w3p, b3r)
  return y3.T[:, 0:4]


# speedup vs baseline: 4.0231x; 1.0545x over previous
import functools

import jax
import jax.numpy as jnp
from jax import lax
from jax.experimental import pallas as pl
from jax.experimental.pallas import tpu as pltpu
from jax.experimental.pallas import tpu_sc as plsc

B = 16384
NC, NS = 2, 16
NW = NC * NS
BPW = B // NW
GSZ = 16
NGRP = BPW // GSZ
TFLAT = 16 * 1024
EPS = 1e-5


def _sc_gather_body(tabc_hbm, tabo_hbm, ic_hbm, io_hbm, out_hbm,
                    tcv, tov, icv, iov, ov, sem):
  wid = lax.axis_index("s") * NC + lax.axis_index("c")
  base = wid * BPW
  cps = [
      pltpu.async_copy(tabc_hbm, tcv, sem),
      pltpu.async_copy(tabo_hbm, tov, sem),
      pltpu.async_copy(ic_hbm.at[pl.ds(base, BPW)], icv, sem),
      pltpu.async_copy(io_hbm.at[pl.ds(base, BPW)], iov, sem),
  ]
  for cp in cps:
    cp.wait()

  @plsc.parallel_loop(0, NGRP, unroll=4)
  def _group(g):
    sl = pl.ds(g * GSZ, GSZ)
    pc = icv[sl]
    po = iov[sl]
    for d in range(10):
      ov[d, sl] = plsc.load_gather(tcv, [pc + d * 1024])
      ov[16 + d, sl] = plsc.load_gather(tov, [po + d * 1024])

  zeros = jnp.zeros((GSZ,), jnp.float32)
  for k in range(NGRP):
    sl = pl.ds(k * GSZ, GSZ)
    for d in range(10, 16):
      ov[d, sl] = zeros
      ov[16 + d, sl] = zeros

  pltpu.sync_copy(ov, out_hbm.at[:, pl.ds(base, BPW)])


def _sc_gather(tab_c, tab_o, idx_c, idx_o):
  mesh = plsc.VectorSubcoreMesh(core_axis_name="c", subcore_axis_name="s")
  fn = functools.partial(
      pl.kernel,
      out_type=jax.ShapeDtypeStruct((32, B), jnp.float32),
      mesh=mesh,
      scratch_types=[
          pltpu.VMEM((TFLAT,), jnp.float32),
          pltpu.VMEM((TFLAT,), jnp.float32),
          pltpu.VMEM((BPW,), jnp.int32),
          pltpu.VMEM((BPW,), jnp.int32),
          pltpu.VMEM((32, BPW), jnp.float32),
          pltpu.SemaphoreType.DMA,
      ],
      compiler_params=pltpu.CompilerParams(needs_layout_passes=False),
  )(_sc_gather_body)
  return fn(tab_c, tab_o, idx_c, idx_o)


def _tc_dense_body(co_ref, tt_ref, gc, bc, go, bo, gt, bt,
                   w1c, w1o, w1t, b1, w2, b2, w3, b3, out_ref):
  inv_b = 1.0 / B

  def bn(xp, g, b):
    m = jnp.sum(xp, axis=1, keepdims=True) * inv_b
    d = xp - m
    v = jnp.sum(d * d, axis=1, keepdims=True) * inv_b
    s = g[...] * lax.rsqrt(v + EPS)
    return d * s + b[...]

  co = co_ref[...]
  xc = bn(co[0:16], gc, bc)
  xo = bn(co[16:32], go, bo)
  xt = bn(tt_ref[...], gt, bt)
  y1 = (jnp.dot(w1c[...], xc, preferred_element_type=jnp.float32)
        + jnp.dot(w1o[...], xo, preferred_element_type=jnp.float32)
        + jnp.dot(w1t[...], xt, preferred_element_type=jnp.float32)
        + b1[...])
  h1 = jnp.maximum(y1, 0.0)
  h2 = jnp.maximum(
      jnp.dot(w2[...], h1, preferred_element_type=jnp.float32) + b2[...], 0.0)
  out_ref[...] = jnp.dot(w3[...], h2, preferred_element_type=jnp.float32) + b3[...]


def _tc_dense(co, tt, gc, bc, go, bo, gt, bt,
              w1c, w1o, w1t, b1, w2, b2, w3, b3, interpret=False):
  return pl.pallas_call(
      _tc_dense_body,
      out_shape=jax.ShapeDtypeStruct((8, B), jnp.float32),
      interpret=interpret,
  )(co, tt, gc, bc, go, bo, gt, bt, w1c, w1o, w1t, b1, w2, b2, w3, b3)


def _pad2(a, rows, cols):
  r, c = a.shape
  return jnp.pad(a, ((0, rows - r), (0, cols - c)))


def _flat_table(tab):
  return jnp.pad(tab.T, ((0, 6), (0, 24))).reshape(TFLAT)


def kernel(x, country_table, occupation_table, bn_gamma, bn_beta,
           W1, b1, W2, b2, W3, b3):
  xt_all = x.T
  idx_c = xt_all[1].astype(jnp.int32)
  idx_o = xt_all[8].astype(jnp.int32)
  other_t = jnp.concatenate(
      [xt_all[0:1], xt_all[3:8], jnp.zeros((2, B), jnp.float32)], axis=0)

  tab_c = _flat_table(country_table)
  tab_o = _flat_table(occupation_table)

  gc = _pad2(bn_gamma[0:10].reshape(10, 1), 16, 1)
  go = _pad2(bn_gamma[10:20].reshape(10, 1), 16, 1)
  gt = _pad2(bn_gamma[20:26].reshape(6, 1), 8, 1)
  bc = _pad2(bn_beta[0:10].reshape(10, 1), 16, 1)
  bo = _pad2(bn_beta[10:20].reshape(10, 1), 16, 1)
  bt = _pad2(bn_beta[20:26].reshape(6, 1), 8, 1)

  w1c = _pad2(W1[:, 0:10], 32, 16)
  w1o = _pad2(W1[:, 10:20], 32, 16)
  w1t = _pad2(W1[:, 20:26], 32, 8)
  w3p = _pad2(W3, 8, 16)
  b1r = b1.reshape(32, 1)
  b2r = b2.reshape(16, 1)
  b3r = _pad2(b3.reshape(4, 1), 8, 1)

  co = _sc_gather(tab_c, tab_o, idx_c, idx_o)
  y3 = _tc_dense(co, other_t, gc, bc, go, bo, gt, bt,
                 w1c, w1o, w1t, b1r, W2, b2r, w3p, b3r)
  return y3.T[:, 0:4]


# speedup vs baseline: 4.3510x; 1.0815x over previous
import functools

import jax
import jax.numpy as jnp
from jax import lax
from jax.experimental import pallas as pl
from jax.experimental.pallas import tpu as pltpu
from jax.experimental.pallas import tpu_sc as plsc

B = 16384
NC, NS = 2, 16
NW = NC * NS
BPW = B // NW
GSZ = 16
NGRP = BPW // GSZ
TFLAT = 16 * 1024
EPS = 1e-5


def _sc_gather_body(tabc_hbm, tabo_hbm, ic_hbm, io_hbm, out_hbm,
                    tcv, tov, icv, iov, ov, sem):
  wid = lax.axis_index("s") * NC + lax.axis_index("c")
  base = wid * BPW
  cps = [
      pltpu.async_copy(tabc_hbm, tcv, sem),
      pltpu.async_copy(tabo_hbm, tov, sem),
      pltpu.async_copy(ic_hbm.at[pl.ds(base, BPW)], icv, sem),
      pltpu.async_copy(io_hbm.at[pl.ds(base, BPW)], iov, sem),
  ]
  for cp in cps:
    cp.wait()

  @plsc.parallel_loop(0, NGRP, unroll=4)
  def _group(g):
    sl = pl.ds(g * GSZ, GSZ)
    pc = icv[sl]
    po = iov[sl]
    for d in range(10):
      ov[d, sl] = plsc.load_gather(tcv, [pc + d * 1024])
      ov[16 + d, sl] = plsc.load_gather(tov, [po + d * 1024])

  zeros = jnp.zeros((GSZ,), jnp.float32)
  for k in range(NGRP):
    sl = pl.ds(k * GSZ, GSZ)
    for d in range(10, 16):
      ov[d, sl] = zeros
      ov[16 + d, sl] = zeros

  pltpu.sync_copy(ov, out_hbm.at[:, pl.ds(base, BPW)])


def _sc_gather(tab_c, tab_o, idx_c, idx_o):
  mesh = plsc.VectorSubcoreMesh(core_axis_name="c", subcore_axis_name="s")
  fn = functools.partial(
      pl.kernel,
      out_type=jax.ShapeDtypeStruct((32, B), jnp.float32),
      mesh=mesh,
      scratch_types=[
          pltpu.VMEM((TFLAT,), jnp.float32),
          pltpu.VMEM((TFLAT,), jnp.float32),
          pltpu.VMEM((BPW,), jnp.int32),
          pltpu.VMEM((BPW,), jnp.int32),
          pltpu.VMEM((32, BPW), jnp.float32),
          pltpu.SemaphoreType.DMA,
      ],
      compiler_params=pltpu.CompilerParams(needs_layout_passes=False),
  )(_sc_gather_body)
  return fn(tab_c, tab_o, idx_c, idx_o)


def _tc_dense_body(co_ref, tt_ref, p_ref, bias_ref, w1p_ref, w1t_ref,
                   w2_ref, w3_ref, out_ref):
  inv_b = 1.0 / B

  def bn(xp, g, b):
    m = jnp.sum(xp, axis=1, keepdims=True) * inv_b
    d = xp - m
    v = jnp.sum(d * d, axis=1, keepdims=True) * inv_b
    s = g * lax.rsqrt(v + EPS)
    return d * s + b

  p = p_ref[...]
  bias = bias_ref[...]
  co = co_ref[...]
  xc = bn(co[0:16], p[:, 0:1], p[:, 1:2])
  xo = bn(co[16:32], p[:, 2:3], p[:, 3:4])
  xt = bn(tt_ref[...], p[0:8, 4:5], p[0:8, 5:6])
  w1p = w1p_ref[...]
  y1 = (jnp.dot(w1p[:, 0:16], xc, preferred_element_type=jnp.float32)
        + jnp.dot(w1p[:, 16:32], xo, preferred_element_type=jnp.float32)
        + jnp.dot(w1t_ref[...], xt, preferred_element_type=jnp.float32)
        + bias[:, 0:1])
  h1 = jnp.maximum(y1, 0.0)
  h2 = jnp.maximum(
      jnp.dot(w2_ref[...], h1, preferred_element_type=jnp.float32)
      + bias[0:16, 1:2], 0.0)
  out_ref[...] = (jnp.dot(w3_ref[...], h2, preferred_element_type=jnp.float32)
                  + bias[0:4, 2:3])


def _tc_dense(co, tt, p, bias, w1p, w1t, w2, w3, interpret=False):
  return pl.pallas_call(
      _tc_dense_body,
      out_shape=jax.ShapeDtypeStruct((4, B), jnp.float32),
      interpret=interpret,
  )(co, tt, p, bias, w1p, w1t, w2, w3)


def _pad2(a, rows, cols):
  r, c = a.shape
  return jnp.pad(a, ((0, rows - r), (0, cols - c)))


def _flat_table(tab):
  return jnp.pad(tab.T, ((0, 6), (0, 24))).reshape(TFLAT)


def kernel(x, country_table, occupation_table, bn_gamma, bn_beta,
           W1, b1, W2, b2, W3, b3):
  xt_all = x.T
  idx_c = xt_all[1].astype(jnp.int32)
  idx_o = xt_all[8].astype(jnp.int32)
  other_t = jnp.concatenate(
      [xt_all[0:1], xt_all[3:8], jnp.zeros((2, B), jnp.float32)], axis=0)

  tab_c = _flat_table(country_table)
  tab_o = _flat_table(occupation_table)

  z6 = jnp.zeros((6,), jnp.float32)
  p = jnp.stack([
      jnp.concatenate([bn_gamma[0:10], z6]),
      jnp.concatenate([bn_beta[0:10], z6]),
      jnp.concatenate([bn_gamma[10:20], z6]),
      jnp.concatenate([bn_beta[10:20], z6]),
      jnp.concatenate([bn_gamma[20:26], jnp.zeros((10,), jnp.float32)]),
      jnp.concatenate([bn_beta[20:26], jnp.zeros((10,), jnp.float32)]),
      jnp.zeros((16,), jnp.float32),
      jnp.zeros((16,), jnp.float32),
  ], axis=1)
  zb = jnp.zeros((32,), jnp.float32)
  bias = jnp.stack([
      b1,
      jnp.concatenate([b2, jnp.zeros((16,), jnp.float32)]),
      jnp.concatenate([b3, jnp.zeros((28,), jnp.float32)]),
      zb, zb, zb, zb, zb,
  ], axis=1)
  zw = jnp.zeros((32, 6), jnp.float32)
  w1p = jnp.concatenate([W1[:, 0:10], zw, W1[:, 10:20], zw], axis=1)
  w1t = _pad2(W1[:, 20:26], 32, 8)

  co = _sc_gather(tab_c, tab_o, idx_c, idx_o)
  y3 = _tc_dense(co, other_t, p, bias, w1p, w1t, W2, W3)
  return y3.T


# speedup vs baseline: 4.9659x; 1.1413x over previous
import functools

import jax
import jax.numpy as jnp
from jax import lax
from jax.experimental import pallas as pl
from jax.experimental.pallas import tpu as pltpu
from jax.experimental.pallas import tpu_sc as plsc

B = 16384
NC, NS = 2, 16
NW = NC * NS
BPT = B // (NW // 2)
GSZ = 16
TFLAT = 10 * 1024
EPS = 1e-5


def _sc_gather_body(tabc_hbm, tabo_hbm, ic_hbm, io_hbm, out_hbm,
                    tv, iv, ov, sem):
  wid = lax.axis_index("s") * NC + lax.axis_index("c")
  half = wid // (NW // 2)
  base = (wid % (NW // 2)) * BPT

  @pl.when(half == 0)
  def _():
    c1 = pltpu.async_copy(tabc_hbm, tv, sem)
    c2 = pltpu.async_copy(ic_hbm.at[pl.ds(base, BPT)], iv, sem)
    c1.wait()
    c2.wait()

  @pl.when(half == 1)
  def _():
    c1 = pltpu.async_copy(tabo_hbm, tv, sem)
    c2 = pltpu.async_copy(io_hbm.at[pl.ds(base, BPT)], iv, sem)
    c1.wait()
    c2.wait()

  @plsc.parallel_loop(0, BPT // GSZ, unroll=4)
  def _group(g):
    sl = pl.ds(g * GSZ, GSZ)
    pc = iv[sl]
    for d in range(10):
      ov[d, sl] = plsc.load_gather(tv, [pc + d * 1024])

  zeros = jnp.zeros((GSZ,), jnp.float32)
  for k in range(BPT // GSZ):
    sl = pl.ds(k * GSZ, GSZ)
    for d in range(10, 16):
      ov[d, sl] = zeros

  pltpu.sync_copy(ov, out_hbm.at[pl.ds(half * 16, 16), pl.ds(base, BPT)])


def _sc_gather(tab_c, tab_o, idx_c, idx_o):
  mesh = plsc.VectorSubcoreMesh(core_axis_name="c", subcore_axis_name="s")
  fn = functools.partial(
      pl.kernel,
      out_type=jax.ShapeDtypeStruct((32, B), jnp.float32),
      mesh=mesh,
      scratch_types=[
          pltpu.VMEM((TFLAT,), jnp.float32),
          pltpu.VMEM((BPT,), jnp.int32),
          pltpu.VMEM((16, BPT), jnp.float32),
          pltpu.SemaphoreType.DMA,
      ],
      compiler_params=pltpu.CompilerParams(needs_layout_passes=False),
  )(_sc_gather_body)
  return fn(tab_c, tab_o, idx_c, idx_o)


def _tc_dense_body(co_ref, tt_ref, p_ref, bias_ref, w1p_ref, w1t_ref,
                   w2_ref, w3_ref, out_ref):
  inv_b = 1.0 / B

  def bn(xp, g, b):
    m = jnp.sum(xp, axis=1, keepdims=True) * inv_b
    d = xp - m
    v = jnp.sum(d * d, axis=1, keepdims=True) * inv_b
    s = g * lax.rsqrt(v + EPS)
    return d * s + b

  p = p_ref[...]
  bias = bias_ref[...]
  co = co_ref[...]
  xc = bn(co[0:16], p[:, 0:1], p[:, 1:2])
  xo = bn(co[16:32], p[:, 2:3], p[:, 3:4])
  xt = bn(tt_ref[...], p[0:8, 4:5], p[0:8, 5:6])
  w1p = w1p_ref[...]
  y1 = (jnp.dot(w1p[:, 0:16], xc, preferred_element_type=jnp.float32)
        + jnp.dot(w1p[:, 16:32], xo, preferred_element_type=jnp.float32)
        + jnp.dot(w1t_ref[...], xt, preferred_element_type=jnp.float32)
        + bias[:, 0:1])
  h1 = jnp.maximum(y1, 0.0)
  h2 = jnp.maximum(
      jnp.dot(w2_ref[...], h1, preferred_element_type=jnp.float32)
      + bias[0:16, 1:2], 0.0)
  out_ref[...] = (jnp.dot(w3_ref[...], h2, preferred_element_type=jnp.float32)
                  + bias[0:4, 2:3])


def _tc_dense(co, tt, p, bias, w1p, w1t, w2, w3, interpret=False):
  return pl.pallas_call(
      _tc_dense_body,
      out_shape=jax.ShapeDtypeStruct((4, B), jnp.float32),
      interpret=interpret,
  )(co, tt, p, bias, w1p, w1t, w2, w3)


def _pad2(a, rows, cols):
  r, c = a.shape
  return jnp.pad(a, ((0, rows - r), (0, cols - c)))


def _flat_table(tab):
  return jnp.pad(tab.T, ((0, 0), (0, 24))).reshape(TFLAT)


def kernel(x, country_table, occupation_table, bn_gamma, bn_beta,
           W1, b1, W2, b2, W3, b3):
  xt_all = x.T
  idx_c = xt_all[1].astype(jnp.int32)
  idx_o = xt_all[8].astype(jnp.int32)
  other_t = jnp.concatenate(
      [xt_all[0:1], xt_all[3:8], jnp.zeros((2, B), jnp.float32)], axis=0)

  tab_c = _flat_table(country_table)
  tab_o = _flat_table(occupation_table)

  z6 = jnp.zeros((6,), jnp.float32)
  p = jnp.stack([
      jnp.concatenate([bn_gamma[0:10], z6]),
      jnp.concatenate([bn_beta[0:10], z6]),
      jnp.concatenate([bn_gamma[10:20], z6]),
      jnp.concatenate([bn_beta[10:20], z6]),
      jnp.concatenate([bn_gamma[20:26], jnp.zeros((10,), jnp.float32)]),
      jnp.concatenate([bn_beta[20:26], jnp.zeros((10,), jnp.float32)]),
      jnp.zeros((16,), jnp.float32),
      jnp.zeros((16,), jnp.float32),
  ], axis=1)
  zb = jnp.zeros((32,), jnp.float32)
  bias = jnp.stack([
      b1,
      jnp.concatenate([b2, jnp.zeros((16,), jnp.float32)]),
      jnp.concatenate([b3, jnp.zeros((28,), jnp.float32)]),
      zb, zb, zb, zb, zb,
  ], axis=1)
  zw = jnp.zeros((32, 6), jnp.float32)
  w1p = jnp.concatenate([W1[:, 0:10], zw, W1[:, 10:20], zw], axis=1)
  w1t = _pad2(W1[:, 20:26], 32, 8)

  co = _sc_gather(tab_c, tab_o, idx_c, idx_o)
  y3 = _tc_dense(co, other_t, p, bias, w1p, w1t, W2, W3)
  return y3.T


# speedup vs baseline: 5.0633x; 1.0196x over previous
import functools

import jax
import jax.numpy as jnp
from jax import lax
from jax.experimental import pallas as pl
from jax.experimental.pallas import tpu as pltpu
from jax.experimental.pallas import tpu_sc as plsc

B = 16384
NC, NS = 2, 16
NW = NC * NS
BPT = B // (NW // 2)
GSZ = 16
TFLAT = 10 * 1024
EPS = 1e-5
OTHER_COLS = (0, 3, 4, 5, 6, 7)


def _sc_gather_body(tabs_hbm, iall_hbm, out_hbm, tv, iv, ov, sem):
  wid = lax.axis_index("s") * NC + lax.axis_index("c")
  half = wid // (NW // 2)
  base = (wid % (NW // 2)) * BPT

  c1 = pltpu.async_copy(tabs_hbm.at[pl.ds(half * TFLAT, TFLAT)], tv, sem)
  c2 = pltpu.async_copy(iall_hbm.at[half, pl.ds(base, BPT)], iv, sem)
  c1.wait()
  c2.wait()

  @plsc.parallel_loop(0, BPT // GSZ, unroll=4)
  def _group(g):
    sl = pl.ds(g * GSZ, GSZ)
    pc = iv[sl]
    for d in range(10):
      ov[d, sl] = plsc.load_gather(tv, [pc + d * 1024])

  zeros = jnp.zeros((GSZ,), jnp.float32)
  for k in range(BPT // GSZ):
    sl = pl.ds(k * GSZ, GSZ)
    for d in range(10, 16):
      ov[d, sl] = zeros

  pltpu.sync_copy(ov, out_hbm.at[pl.ds(half * 16, 16), pl.ds(base, BPT)])


def _sc_gather(tabs, iall):
  mesh = plsc.VectorSubcoreMesh(core_axis_name="c", subcore_axis_name="s")
  fn = functools.partial(
      pl.kernel,
      out_type=jax.ShapeDtypeStruct((32, B), jnp.float32),
      mesh=mesh,
      scratch_types=[
          pltpu.VMEM((TFLAT,), jnp.float32),
          pltpu.VMEM((BPT,), jnp.int32),
          pltpu.VMEM((16, BPT), jnp.float32),
          pltpu.SemaphoreType.DMA,
      ],
      compiler_params=pltpu.CompilerParams(needs_layout_passes=False),
  )(_sc_gather_body)
  return fn(tabs, iall)


def _tc_dense_body(co_ref, xt_ref, pb_ref, out_ref):
  inv_b = 1.0 / B

  def bn(xp, g, b):
    m = jnp.sum(xp, axis=1, keepdims=True) * inv_b
    d = xp - m
    v = jnp.sum(d * d, axis=1, keepdims=True) * inv_b
    s = g * lax.rsqrt(v + EPS)
    return d * s + b

  pb = pb_ref[...]
  p = pb[0:16, 0:8]
  co = co_ref[...]
  xt = xt_ref[...]
  xc = bn(co[0:16], p[:, 0:1], p[:, 1:2])
  xo = bn(co[16:32], p[:, 2:3], p[:, 3:4])
  xn = bn(xt, pb[0:10, 4:5], pb[0:10, 5:6])
  y1 = (jnp.dot(pb[:, 16:32], xc, preferred_element_type=jnp.float32)
        + jnp.dot(pb[:, 32:48], xo, preferred_element_type=jnp.float32)
        + jnp.dot(pb[:, 48:58], xn, preferred_element_type=jnp.float32)
        + pb[:, 8:9])
  h1 = jnp.maximum(y1, 0.0)
  h2 = jnp.maximum(
      jnp.dot(pb[0:16, 64:96], h1, preferred_element_type=jnp.float32)
      + pb[0:16, 9:10], 0.0)
  out_ref[...] = (jnp.dot(pb[0:4, 96:112], h2,
                          preferred_element_type=jnp.float32)
                  + pb[0:4, 10:11])


def _tc_dense(co, xt, pb, interpret=False):
  return pl.pallas_call(
      _tc_dense_body,
      out_shape=jax.ShapeDtypeStruct((4, B), jnp.float32),
      interpret=interpret,
  )(co, xt, pb)


def _flat_table(tab):
  return jnp.pad(tab.T, ((0, 0), (0, 24))).reshape(TFLAT)


def _pack_params(bn_gamma, bn_beta, W1, b1, W2, b2, W3, b3):
  z6 = jnp.zeros((6,), jnp.float32)
  z10 = jnp.zeros((10,), jnp.float32)

  def spread10(v6):
    return jnp.concatenate([v6[0:1], jnp.zeros((2,), jnp.float32), v6[1:6],
                            jnp.zeros((2,), jnp.float32)])

  p6 = jnp.stack([
      jnp.concatenate([bn_gamma[0:10], z6]),
      jnp.concatenate([bn_beta[0:10], z6]),
      jnp.concatenate([bn_gamma[10:20], z6]),
      jnp.concatenate([bn_beta[10:20], z6]),
      jnp.concatenate([spread10(bn_gamma[20:26]), z6]),
      jnp.concatenate([spread10(bn_beta[20:26]), z6]),
      jnp.zeros((16,), jnp.float32),
      jnp.zeros((16,), jnp.float32),
  ], axis=1)
  p6 = jnp.pad(p6, ((0, 16), (0, 0)))
  bcol = jnp.stack([
      b1,
      jnp.concatenate([b2, jnp.zeros((16,), jnp.float32)]),
      jnp.concatenate([b3, jnp.zeros((28,), jnp.float32)]),
  ], axis=1)
  bcol = jnp.pad(bcol, ((0, 0), (0, 5)))
  zc = jnp.zeros((32, 6), jnp.float32)
  z2c = jnp.zeros((32, 2), jnp.float32)
  w1c = jnp.concatenate([W1[:, 0:10], zc], axis=1)
  w1o = jnp.concatenate([W1[:, 10:20], zc], axis=1)
  w1t = jnp.concatenate(
      [W1[:, 20:21], z2c, W1[:, 21:26], z2c, zc], axis=1)
  w2p = jnp.pad(W2, ((0, 16), (0, 0)))
  w3p = jnp.pad(W3, ((0, 28), (0, 0)))
  return jnp.concatenate([p6, bcol, w1c, w1o, w1t, w2p, w3p], axis=1)


def kernel(x, country_table, occupation_table, bn_gamma, bn_beta,
           W1, b1, W2, b2, W3, b3):
  xt_all = x.T
  idx_c = xt_all[1].astype(jnp.int32)
  idx_o = xt_all[8].astype(jnp.int32)
  iall = jnp.stack([idx_c, idx_o], axis=0)
  tabs = jnp.concatenate(
      [_flat_table(country_table), _flat_table(occupation_table)])

  co = _sc_gather(tabs, iall)
  y3 = _tc_dense(co, xt_all[0:10], _pack_params(bn_gamma, bn_beta,
                                                W1, b1, W2, b2, W3, b3))
  return y3.T
